# baseline (speedup 1.0000x reference)
import functools

import jax
import jax.numpy as jnp
from jax import lax
from jax.experimental import pallas as pl
from jax.experimental.pallas import tpu as pltpu
from jax.experimental.pallas import tpu_sc as plsc

N = 16777216
THRESHOLD = 0.5
NC = 2
NS = 16
NW = NC * NS
L = 16
W = N // NW
S = 16384
NSUB = W // S
VECS = S // L
CS = 32768
CNSUB = W // CS
CVECS = CS // L
U = 16
ZS = 16384

_mesh = plsc.VectorSubcoreMesh(
    core_axis_name="c", subcore_axis_name="s", num_cores=NC, num_subcores=NS
)
_params = pltpu.CompilerParams(needs_layout_passes=False)


def _worker_id():
    return lax.axis_index("s") * NC + lax.axis_index("c")


@functools.partial(
    pl.kernel,
    out_type=jax.ShapeDtypeStruct((NW * 8,), jnp.int32),
    mesh=_mesh,
    scratch_types=[
        pltpu.VMEM((CS,), jnp.float32),
        pltpu.VMEM((CS,), jnp.float32),
        pltpu.VMEM((L,), jnp.int32),
        pltpu.SemaphoreType.DMA,
        pltpu.SemaphoreType.DMA,
    ],
    compiler_params=_params,
)
def _count_kernel(x_hbm, counts_hbm, in0, in1, stage_v, sem0, sem1):
    w = _worker_id()
    base = pl.multiple_of(w * W, 8)

    def chunk_src(j):
        return x_hbm.at[pl.ds(base + j * CS, CS)]

    def scan_buf(buf, acc):
        def inner(k, acc):
            for u in range(U):
                off = pl.multiple_of(k * (U * L), 8) + u * L
                xv = buf[pl.ds(off, L)]
                acc = acc + jnp.where(xv >= THRESHOLD, 1, 0)
            return acc

        return lax.fori_loop(0, CVECS // U, inner, acc)

    pltpu.async_copy(chunk_src(0), in0, sem0)
    pltpu.async_copy(chunk_src(1), in1, sem1)

    def sub2(jj, acc):
        j0 = 2 * jj
        pltpu.make_async_copy(chunk_src(j0), in0, sem0).wait()
        acc = scan_buf(in0, acc)

        @pl.when(j0 + 2 < CNSUB)
        def _():
            pltpu.async_copy(chunk_src(j0 + 2), in0, sem0)

        pltpu.make_async_copy(chunk_src(j0 + 1), in1, sem1).wait()
        acc = scan_buf(in1, acc)

        @pl.when(j0 + 3 < CNSUB)
        def _():
            pltpu.async_copy(chunk_src(j0 + 3), in1, sem1)

        return acc

    acc = lax.fori_loop(0, CNSUB // 2, sub2, jnp.zeros((L,), jnp.int32))
    stage_v[...] = jnp.full((L,), jnp.sum(acc), jnp.int32)
    pltpu.sync_copy(
        stage_v.at[pl.ds(0, 8)], counts_hbm.at[pl.ds(pl.multiple_of(w * 8, 8), 8)]
    )


@functools.partial(
    pl.kernel,
    out_type=(
        jax.ShapeDtypeStruct((N,), jnp.float32),
        jax.ShapeDtypeStruct((8,), jnp.int32),
    ),
    mesh=_mesh,
    scratch_types=[
        pltpu.VMEM((S,), jnp.float32),
        pltpu.VMEM((S,), jnp.float32),
        pltpu.VMEM((S + 32,), jnp.float32),
        pltpu.VMEM((S + 32,), jnp.float32),
        pltpu.VMEM((ZS,), jnp.float32),
        pltpu.VMEM((NW * 8,), jnp.int32),
        pltpu.VMEM((L,), jnp.int32),
        pltpu.VMEM((L,), jnp.float32),
        pltpu.VMEM((L,), jnp.int32),
        pltpu.VMEM((L,), jnp.float32),
        pltpu.VMEM((L,), jnp.int32),
        pltpu.SemaphoreType.DMA,
        pltpu.SemaphoreType.DMA,
        pltpu.SemaphoreType.DMA,
        pltpu.SemaphoreType.DMA,
    ],
    compiler_params=_params,
)
def _compact_kernel(
    x_hbm,
    counts_hbm,
    out_hbm,
    cnt_hbm,
    in0,
    in1,
    cmp0,
    cmp1,
    zero_v,
    cnts_v,
    idx_v,
    val_v,
    idx_w,
    val_w,
    st_v,
    sem0,
    sem1,
    csem0,
    csem1,
):
    w = _worker_id()
    base = pl.multiple_of(w * W, 8)
    lane = lax.iota(jnp.int32, L)

    def chunk_src(j):
        return x_hbm.at[pl.ds(base + j * S, S)]

    pltpu.async_copy(chunk_src(0), in0, sem0)
    pltpu.async_copy(chunk_src(1), in1, sem1)

    pltpu.sync_copy(counts_hbm, cnts_v)

    def red(k, accs):
        ao, at = accs
        v = cnts_v[pl.ds(k * L, L)]
        gi = 2 * k + jnp.where(lane >= 8, 1, 0)
        ok = (lane & 7) == 0
        ao = ao + jnp.where(ok & (gi < w), v, 0)
        at = at + jnp.where(ok, v, 0)
        return ao, at

    z16 = jnp.zeros((L,), jnp.int32)
    ao, at = lax.fori_loop(0, NW // 2, red, (z16, z16))
    off = jnp.sum(ao)
    total = jnp.sum(at)

    def zfill(k, _):
        zero_v[pl.ds(k * L, L)] = jnp.zeros((L,), jnp.float32)
        return 0

    lax.fori_loop(0, ZS // L, zfill, 0)

    def scatter16(idx, vals):
        idx_v[...] = idx
        val_v[...] = vals
        pltpu.sync_copy(val_v, out_hbm.at[idx_v])

    def lane0(vals):
        return jnp.sum(jnp.where(lane == 0, vals, 0.0))

    def flush_dmas(cmpb, sem, idxb, valb, mid, la, a, c, fire):
        @pl.when(c > 0)
        def _():
            d = pltpu.make_async_copy(valb, out_hbm.at[idxb], sem)
            d.start() if fire else d.wait()

        ch = S
        while ch >= 8:
            hi = mid & (~(2 * ch - 1))

            @pl.when((mid & ch) != 0)
            def _(ch=ch, hi=hi):
                src = cmpb.at[pl.ds(pl.multiple_of(la + hi, 8), ch)]
                dst = out_hbm.at[pl.ds(pl.multiple_of(a + hi, 8), ch)]
                if fire:
                    pltpu.async_copy(src, dst, sem)
                else:
                    pltpu.make_async_copy(src, dst, sem).wait()

            ch //= 2

    def process(buf, cmpb, sem, idxb, valb, cur):
        ptr0 = cur & 7

        def inner(k, ptrv):
            o = pl.multiple_of(k * L, 8)
            xv = buf[pl.ds(o, L)]
            m = xv >= THRESHOLD
            mi = jnp.where(m, 1, 0)
            rank = plsc.cumsum(mi) - mi
            plsc.store_scatter(cmpb, [ptrv + rank], xv, mask=m)
            return ptrv + plsc.all_reduce_population_count(m)

        ptr_end = plsc.parallel_loop(
            0, VECS, 1, unroll=U, carry=jnp.full((L,), ptr0, jnp.int32)
        )(inner)
        c = jnp.max(ptr_end) - ptr0

        e = cur + c
        a = jnp.minimum(cur + ((-cur) & 7), e)
        b = jnp.maximum(e - (e & 7), a)
        fl8 = cur - ptr0

        hl = a - cur
        tl = e - b
        lb = b - fl8

        @pl.when(c > 0)
        def _():
            li = jnp.where(
                lane < hl,
                ptr0 + lane,
                jnp.where(lane < hl + tl, lb + (lane - hl), ptr0),
            )
            gi = jnp.where(
                lane < hl,
                cur + lane,
                jnp.where(lane < hl + tl, b + (lane - hl), cur),
            )
            idxb[...] = gi
            valb[...] = plsc.load_gather(cmpb, [li])

        mid = b - a
        la = a - fl8
        flush_dmas(cmpb, sem, idxb, valb, mid, la, a, c, fire=True)
        return e, mid, la, a, c

    zero4 = (jnp.int32(0),) * 4

    def sub2(jj, st):
        cur, f0, f1 = st
        j0 = 2 * jj
        pltpu.make_async_copy(chunk_src(j0), in0, sem0).wait()
        flush_dmas(cmp0, csem0, idx_v, val_v, *f0, fire=False)
        cur, *f0 = process(in0, cmp0, csem0, idx_v, val_v, cur)
        f0 = tuple(f0)

        @pl.when(j0 + 2 < NSUB)
        def _():
            pltpu.async_copy(chunk_src(j0 + 2), in0, sem0)

        pltpu.make_async_copy(chunk_src(j0 + 1), in1, sem1).wait()
        flush_dmas(cmp1, csem1, idx_w, val_w, *f1, fire=False)
        cur, *f1 = process(in1, cmp1, csem1, idx_w, val_w, cur)
        f1 = tuple(f1)

        @pl.when(j0 + 3 < NSUB)
        def _():
            pltpu.async_copy(chunk_src(j0 + 3), in1, sem1)

        return cur, f0, f1

    _, f0, f1 = lax.fori_loop(0, NSUB // 2, sub2, (off, zero4, zero4))
    flush_dmas(cmp0, csem0, idx_v, val_v, *f0, fire=False)
    flush_dmas(cmp1, csem1, idx_w, val_w, *f1, fire=False)

    endw = base + W
    z0 = jnp.maximum(total, base)

    @pl.when(endw - z0 > 0)
    def _():
        h8 = (-z0) & 7
        scatter16(
            jnp.where(lane < h8, z0 + lane, z0), jnp.zeros((L,), jnp.float32)
        )
        za = z0 + h8
        zal = endw - za
        nfull = zal // ZS

        def zdma(i, _):
            pltpu.sync_copy(
                zero_v, out_hbm.at[pl.ds(pl.multiple_of(za + i * ZS, 8), ZS)]
            )
            return 0

        lax.fori_loop(0, nfull, zdma, 0)
        rem = zal & (ZS - 1)
        rbase = za + nfull * ZS
        ch = ZS // 2
        while ch >= 8:
            hi = rem & (~(2 * ch - 1))

            @pl.when((rem & ch) != 0)
            def _(ch=ch, hi=hi):
                pltpu.sync_copy(
                    zero_v.at[pl.ds(0, ch)],
                    out_hbm.at[pl.ds(pl.multiple_of(rbase + hi, 8), ch)],
                )

            ch //= 2

    @pl.when(w == 0)
    def _():
        st_v[...] = jnp.full((L,), total, jnp.int32)
        pltpu.sync_copy(st_v.at[pl.ds(0, 8)], cnt_hbm)


@jax.jit
def kernel(input):
    counts = _count_kernel(input)
    out, cnt = _compact_kernel(input, counts)
    return out, cnt[0]


# speedup vs baseline: 1.2401x; 1.2401x over previous
import functools

import jax
import jax.numpy as jnp
from jax import lax
from jax.experimental import pallas as pl
from jax.experimental.pallas import tpu as pltpu
from jax.experimental.pallas import tpu_sc as plsc

N = 16777216
THRESHOLD = 0.5
NC = 2
NS = 16
NW = NC * NS
L = 16
W = N // NW
S = 16384
NSUB = W // S
VECS = S // L
FS = 2 * S
NF = W // FS
CS = 32768
CNSUB = W // CS
CVECS = CS // L
U = 8
ZS = 16384

_mesh = plsc.VectorSubcoreMesh(
    core_axis_name="c", subcore_axis_name="s", num_cores=NC, num_subcores=NS
)
_params = pltpu.CompilerParams(needs_layout_passes=False)


def _worker_id():
    return lax.axis_index("s") * NC + lax.axis_index("c")


@functools.partial(
    pl.kernel,
    out_type=jax.ShapeDtypeStruct((NW * 8,), jnp.int32),
    mesh=_mesh,
    scratch_types=[
        pltpu.VMEM((CS,), jnp.float32),
        pltpu.VMEM((CS,), jnp.float32),
        pltpu.VMEM((L,), jnp.int32),
        pltpu.SemaphoreType.DMA,
        pltpu.SemaphoreType.DMA,
    ],
    compiler_params=_params,
)
def _count_kernel(x_hbm, counts_hbm, in0, in1, stage_v, sem0, sem1):
    w = _worker_id()
    base = pl.multiple_of(w * W, 8)

    def chunk_src(j):
        return x_hbm.at[pl.ds(base + j * CS, CS)]

    def scan_buf(buf, acc):
        def inner(k, acc):
            for u in range(U):
                off = pl.multiple_of(k * (U * L), 8) + u * L
                xv = buf[pl.ds(off, L)]
                acc = acc + jnp.where(xv >= THRESHOLD, 1, 0)
            return acc

        return lax.fori_loop(0, CVECS // U, inner, acc)

    pltpu.async_copy(chunk_src(0), in0, sem0)
    pltpu.async_copy(chunk_src(1), in1, sem1)

    def sub2(jj, acc):
        j0 = 2 * jj
        pltpu.make_async_copy(chunk_src(j0), in0, sem0).wait()
        acc = scan_buf(in0, acc)

        @pl.when(j0 + 2 < CNSUB)
        def _():
            pltpu.async_copy(chunk_src(j0 + 2), in0, sem0)

        pltpu.make_async_copy(chunk_src(j0 + 1), in1, sem1).wait()
        acc = scan_buf(in1, acc)

        @pl.when(j0 + 3 < CNSUB)
        def _():
            pltpu.async_copy(chunk_src(j0 + 3), in1, sem1)

        return acc

    acc = lax.fori_loop(0, CNSUB // 2, sub2, jnp.zeros((L,), jnp.int32))
    stage_v[...] = jnp.full((L,), jnp.sum(acc), jnp.int32)
    pltpu.sync_copy(
        stage_v.at[pl.ds(0, 8)], counts_hbm.at[pl.ds(pl.multiple_of(w * 8, 8), 8)]
    )


@functools.partial(
    pl.kernel,
    out_type=(
        jax.ShapeDtypeStruct((N,), jnp.float32),
        jax.ShapeDtypeStruct((8,), jnp.int32),
    ),
    mesh=_mesh,
    scratch_types=[
        pltpu.VMEM((S,), jnp.float32),
        pltpu.VMEM((S,), jnp.float32),
        pltpu.VMEM((FS + 32,), jnp.float32),
        pltpu.VMEM((FS + 32,), jnp.float32),
        pltpu.VMEM((ZS,), jnp.float32),
        pltpu.VMEM((NW * 8,), jnp.int32),
        pltpu.VMEM((L,), jnp.int32),
        pltpu.VMEM((L,), jnp.float32),
        pltpu.VMEM((L,), jnp.int32),
        pltpu.VMEM((L,), jnp.float32),
        pltpu.VMEM((L,), jnp.int32),
        pltpu.SemaphoreType.DMA,
        pltpu.SemaphoreType.DMA,
        pltpu.SemaphoreType.DMA,
        pltpu.SemaphoreType.DMA,
    ],
    compiler_params=_params,
)
def _compact_kernel(
    x_hbm,
    counts_hbm,
    out_hbm,
    cnt_hbm,
    in0,
    in1,
    cmp0,
    cmp1,
    zero_v,
    cnts_v,
    idx_v,
    val_v,
    idx_w,
    val_w,
    st_v,
    sem0,
    sem1,
    csem0,
    csem1,
):
    w = _worker_id()
    base = pl.multiple_of(w * W, 8)
    lane = lax.iota(jnp.int32, L)

    def chunk_src(j):
        return x_hbm.at[pl.ds(base + j * S, S)]

    pltpu.async_copy(chunk_src(0), in0, sem0)
    pltpu.async_copy(chunk_src(1), in1, sem1)

    pltpu.sync_copy(counts_hbm, cnts_v)

    def red(k, accs):
        ao, at = accs
        v = cnts_v[pl.ds(k * L, L)]
        gi = 2 * k + jnp.where(lane >= 8, 1, 0)
        ok = (lane & 7) == 0
        ao = ao + jnp.where(ok & (gi < w), v, 0)
        at = at + jnp.where(ok, v, 0)
        return ao, at

    z16 = jnp.zeros((L,), jnp.int32)
    ao, at = lax.fori_loop(0, NW // 2, red, (z16, z16))
    off = jnp.sum(ao)
    total = jnp.sum(at)

    def zfill(k, _):
        zero_v[pl.ds(k * L, L)] = jnp.zeros((L,), jnp.float32)
        return 0

    lax.fori_loop(0, ZS // L, zfill, 0)

    def scatter16(idx, vals):
        idx_v[...] = idx
        val_v[...] = vals
        pltpu.sync_copy(val_v, out_hbm.at[idx_v])

    def lane0(vals):
        return jnp.sum(jnp.where(lane == 0, vals, 0.0))

    def flush_dmas(cmpb, sem, idxb, valb, mid, la, a, c, fire):
        @pl.when(c > 0)
        def _():
            d = pltpu.make_async_copy(valb, out_hbm.at[idxb], sem)
            d.start() if fire else d.wait()

        ch = FS
        while ch >= 8:
            hi = mid & (~(2 * ch - 1))

            @pl.when((mid & ch) != 0)
            def _(ch=ch, hi=hi):
                src = cmpb.at[pl.ds(pl.multiple_of(la + hi, 8), ch)]
                dst = out_hbm.at[pl.ds(pl.multiple_of(a + hi, 8), ch)]
                if fire:
                    pltpu.async_copy(src, dst, sem)
                else:
                    pltpu.make_async_copy(src, dst, sem).wait()

            ch //= 2

    def process(j, cmpb, sem, idxb, valb, cur):
        ptr0 = cur & 7

        def compact_buf(buf, ptrv):
            def inner(k, ptrv):
                o = pl.multiple_of(k * L, 8)
                xv = buf[pl.ds(o, L)]
                m = xv >= THRESHOLD
                mi = jnp.where(m, 1, 0)
                rank = plsc.cumsum(mi) - mi
                plsc.store_scatter(cmpb, [ptrv + rank], xv, mask=m)
                return ptrv + plsc.all_reduce_population_count(m)

            return plsc.parallel_loop(0, VECS, 1, unroll=U, carry=ptrv)(inner)

        pltpu.make_async_copy(chunk_src(j), in0, sem0).wait()
        ptrv = compact_buf(in0, jnp.full((L,), ptr0, jnp.int32))

        @pl.when(j + 2 < NSUB)
        def _():
            pltpu.async_copy(chunk_src(j + 2), in0, sem0)

        pltpu.make_async_copy(chunk_src(j + 1), in1, sem1).wait()
        ptrv = compact_buf(in1, ptrv)

        @pl.when(j + 3 < NSUB)
        def _():
            pltpu.async_copy(chunk_src(j + 3), in1, sem1)

        c = jnp.max(ptrv) - ptr0

        e = cur + c
        a = jnp.minimum(cur + ((-cur) & 7), e)
        b = jnp.maximum(e - (e & 7), a)
        fl8 = cur - ptr0

        hl = a - cur
        tl = e - b
        lb = b - fl8

        @pl.when(c > 0)
        def _():
            li = jnp.where(
                lane < hl,
                ptr0 + lane,
                jnp.where(lane < hl + tl, lb + (lane - hl), ptr0),
            )
            gi = jnp.where(
                lane < hl,
                cur + lane,
                jnp.where(lane < hl + tl, b + (lane - hl), cur),
            )
            idxb[...] = gi
            valb[...] = plsc.load_gather(cmpb, [li])

        mid = b - a
        la = a - fl8
        flush_dmas(cmpb, sem, idxb, valb, mid, la, a, c, fire=True)
        return e, mid, la, a, c

    zero4 = (jnp.int32(0),) * 4

    def sub2(ff, st):
        cur, f0, f1 = st
        j0 = 4 * ff
        flush_dmas(cmp0, csem0, idx_v, val_v, *f0, fire=False)
        cur, *f0 = process(j0, cmp0, csem0, idx_v, val_v, cur)
        f0 = tuple(f0)
        flush_dmas(cmp1, csem1, idx_w, val_w, *f1, fire=False)
        cur, *f1 = process(j0 + 2, cmp1, csem1, idx_w, val_w, cur)
        f1 = tuple(f1)
        return cur, f0, f1

    _, f0, f1 = lax.fori_loop(0, NF // 2, sub2, (off, zero4, zero4))
    flush_dmas(cmp0, csem0, idx_v, val_v, *f0, fire=False)
    flush_dmas(cmp1, csem1, idx_w, val_w, *f1, fire=False)

    endw = base + W
    z0 = jnp.maximum(total, base)

    @pl.when(endw - z0 > 0)
    def _():
        h8 = (-z0) & 7
        scatter16(
            jnp.where(lane < h8, z0 + lane, z0), jnp.zeros((L,), jnp.float32)
        )
        za = z0 + h8
        zal = endw - za
        nfull = zal // ZS

        def zdma(i, _):
            pltpu.sync_copy(
                zero_v, out_hbm.at[pl.ds(pl.multiple_of(za + i * ZS, 8), ZS)]
            )
            return 0

        lax.fori_loop(0, nfull, zdma, 0)
        rem = zal & (ZS - 1)
        rbase = za + nfull * ZS
        ch = ZS // 2
        while ch >= 8:
            hi = rem & (~(2 * ch - 1))

            @pl.when((rem & ch) != 0)
            def _(ch=ch, hi=hi):
                pltpu.sync_copy(
                    zero_v.at[pl.ds(0, ch)],
                    out_hbm.at[pl.ds(pl.multiple_of(rbase + hi, 8), ch)],
                )

            ch //= 2

    @pl.when(w == 0)
    def _():
        st_v[...] = jnp.full((L,), total, jnp.int32)
        pltpu.sync_copy(st_v.at[pl.ds(0, 8)], cnt_hbm)


@jax.jit
def kernel(input):
    counts = _count_kernel(input)
    out, cnt = _compact_kernel(input, counts)
    return out, cnt[0]
